# baseline (speedup 1.0000x reference)
import functools

import jax
import jax.numpy as jnp
from jax.experimental import pallas as pl
from jax.experimental.pallas import tpu as pltpu

_TILE = 16384
_SPLIT_NUM = 376
_SPLIT_DEN = 512


def _native_kernel(d_hbm, s_hbm, m_hbm, out_ref, d_buf, s_buf, m_buf,
                   acc_l, acc_m, sem, *, total_rows, rows, tile, minor):
    steps = rows // tile
    rem = rows - steps * tile
    hbms = (d_hbm.reshape(total_rows, minor), s_hbm.reshape(total_rows, minor),
            m_hbm.reshape(total_rows, minor))
    bufs = (d_buf, s_buf, m_buf)

    def dma(slot, step, nrows):
        for k in range(3):
            pltpu.make_async_copy(
                hbms[k].at[pl.ds(step * tile, nrows)],
                bufs[k].at[slot, pl.ds(0, nrows)],
                sem.at[k, slot]).start()

    def wait(slot, nrows):
        for k in range(3):
            pltpu.make_async_copy(
                hbms[k].at[pl.ds(0, nrows)],
                bufs[k].at[slot, pl.ds(0, nrows)],
                sem.at[k, slot]).wait()

    def accumulate(d, s, m):
        l = (jnp.abs(d) + s) * m
        acc_l[...] += l.reshape(-1, 8, minor).sum(axis=0)
        acc_m[...] += m.reshape(-1, 8, minor).sum(axis=0)

    acc_l[...] = jnp.zeros_like(acc_l)
    acc_m[...] = jnp.zeros_like(acc_m)

    if steps > 0:
        dma(0, 0, tile)

        def body(j, _):
            cur = jax.lax.rem(j, 2)
            nxt = jax.lax.rem(j + 1, 2)

            @pl.when(j + 1 < steps)
            def _():
                dma(nxt, j + 1, tile)

            wait(cur, tile)
            accumulate(d_buf[cur], s_buf[cur], m_buf[cur])
            return ()

        jax.lax.fori_loop(0, steps, body, (), unroll=False)

    if rem:
        tslot = steps % 2
        dma(tslot, steps, rem)
        wait(tslot, rem)
        d = d_buf[tslot, :rem]
        s = s_buf[tslot, :rem]
        m = m_buf[tslot, :rem]
        pad = (-rem) % 8
        if pad:
            z = jnp.zeros((pad, minor), jnp.float32)
            d = jnp.concatenate([d, z], 0)
            s = jnp.concatenate([s, z], 0)
            m = jnp.concatenate([m, z], 0)
        accumulate(d, s, m)

    out_ref[0, 0] = acc_l[...].sum()
    out_ref[0, 1] = acc_m[...].sum()


def _dense_tail_kernel(part_ref, d_ref, s_ref, m_ref, out_ref):
    l = (jnp.abs(d_ref[...]) + s_ref[...]) * m_ref[...]
    sum_l = part_ref[0, 0] + l.sum()
    sum_m = part_ref[0, 1] + m_ref[...].sum()
    out_ref[0, 0] = sum_l / sum_m


def _native_sums(delta_norm, logstd, M_obs, rows, minor):
    total_rows = delta_norm.size // minor
    tile = min(_TILE, max(8, (rows // 8) * 8))
    return pl.pallas_call(
        functools.partial(_native_kernel, total_rows=total_rows, rows=rows,
                          tile=tile, minor=minor),
        out_shape=jax.ShapeDtypeStruct((1, 2), jnp.float32),
        in_specs=[pl.BlockSpec(memory_space=pltpu.MemorySpace.HBM)] * 3,
        out_specs=pl.BlockSpec(memory_space=pltpu.SMEM),
        scratch_shapes=[pltpu.VMEM((2, tile, minor), jnp.float32),
                        pltpu.VMEM((2, tile, minor), jnp.float32),
                        pltpu.VMEM((2, tile, minor), jnp.float32),
                        pltpu.VMEM((8, minor), jnp.float32),
                        pltpu.VMEM((8, minor), jnp.float32),
                        pltpu.SemaphoreType.DMA((3, 2))],
        cost_estimate=pl.CostEstimate(
            flops=int(5 * rows * minor), transcendentals=0,
            bytes_accessed=int(12 * rows * minor)),
    )(delta_norm, logstd, M_obs)


def kernel(delta_norm, logstd, M_obs):
    f32 = jnp.float32
    if delta_norm.ndim < 2:
        delta_norm = delta_norm.reshape(1, -1)
        logstd = logstd.reshape(1, -1)
        M_obs = M_obs.reshape(1, -1)
    shape = delta_norm.shape
    minor = shape[-1]
    b0 = shape[0]
    per_major = delta_norm.size // b0

    split = (b0 * _SPLIT_NUM) // _SPLIT_DEN
    tail_elems = (b0 - split) * per_major
    if per_major % 128 or split == 0 or tail_elems == 0 \
            or tail_elems // 128 > 32768 or (tail_elems // 128) % 8:
        split = b0
        tail_elems = 0

    rows_native = (split * per_major) // minor
    part = _native_sums(delta_norm, logstd, M_obs, rows_native, minor)

    if tail_elems == 0:
        return (part[0, 0] / part[0, 1]).astype(f32)

    tail_rows = tail_elems // 128
    dd = delta_norm[split:].reshape(tail_rows, 128)
    sd = logstd[split:].reshape(tail_rows, 128)
    md = M_obs[split:].reshape(tail_rows, 128)

    full = pl.BlockSpec((tail_rows, 128), lambda: (0, 0))
    out = pl.pallas_call(
        _dense_tail_kernel,
        out_shape=jax.ShapeDtypeStruct((1, 1), f32),
        in_specs=[pl.BlockSpec(memory_space=pltpu.SMEM), full, full, full],
        out_specs=pl.BlockSpec(memory_space=pltpu.SMEM),
        cost_estimate=pl.CostEstimate(
            flops=int(5 * tail_elems), transcendentals=0,
            bytes_accessed=int(12 * tail_elems)),
    )(part, dd, sd, md)
    return out[0, 0]


# speedup vs baseline: 1.3476x; 1.3476x over previous
import functools

import jax
import jax.numpy as jnp
from jax.experimental import pallas as pl
from jax.experimental.pallas import tpu as pltpu

_TILE = 16384


def _sums_kernel(d_hbm, s_hbm, m_hbm, out_ref, d_buf, s_buf, m_buf,
                 acc_l, acc_m, sem, *, rows, tile, minor):
    steps = rows // tile
    rem = rows - steps * tile
    hbms = (d_hbm.reshape(rows, minor), s_hbm.reshape(rows, minor),
            m_hbm.reshape(rows, minor))
    bufs = (d_buf, s_buf, m_buf)

    def dma(slot, step, nrows):
        for k in range(3):
            pltpu.make_async_copy(
                hbms[k].at[pl.ds(step * tile, nrows)],
                bufs[k].at[slot, pl.ds(0, nrows)],
                sem.at[k, slot]).start()

    def wait(slot, nrows):
        for k in range(3):
            pltpu.make_async_copy(
                hbms[k].at[pl.ds(0, nrows)],
                bufs[k].at[slot, pl.ds(0, nrows)],
                sem.at[k, slot]).wait()

    def accumulate(d, s, m):
        l = (jnp.abs(d) + s) * m
        acc_l[...] += l.reshape(-1, 8, minor).sum(axis=0)
        acc_m[...] += m.reshape(-1, 8, minor).sum(axis=0)

    acc_l[...] = jnp.zeros_like(acc_l)
    acc_m[...] = jnp.zeros_like(acc_m)

    if steps > 0:
        dma(0, 0, tile)

        def body(j, _):
            cur = jax.lax.rem(j, 2)
            nxt = jax.lax.rem(j + 1, 2)

            @pl.when(j + 1 < steps)
            def _():
                dma(nxt, j + 1, tile)

            wait(cur, tile)
            accumulate(d_buf[cur], s_buf[cur], m_buf[cur])
            return ()

        jax.lax.fori_loop(0, steps, body, (), unroll=False)

    if rem:
        tslot = steps % 2
        dma(tslot, steps, rem)
        wait(tslot, rem)
        d = d_buf[tslot, :rem]
        s = s_buf[tslot, :rem]
        m = m_buf[tslot, :rem]
        pad = (-rem) % 8
        if pad:
            z = jnp.zeros((pad, minor), jnp.float32)
            d = jnp.concatenate([d, z], 0)
            s = jnp.concatenate([s, z], 0)
            m = jnp.concatenate([m, z], 0)
        accumulate(d, s, m)

    out_ref[0, 0] = acc_l[...].sum() / acc_m[...].sum()


def kernel(delta_norm, logstd, M_obs):
    f32 = jnp.float32
    if delta_norm.ndim < 2:
        delta_norm = delta_norm.reshape(1, -1)
        logstd = logstd.reshape(1, -1)
        M_obs = M_obs.reshape(1, -1)
    minor = delta_norm.shape[-1]
    rows = delta_norm.size // minor
    tile = min(_TILE, max(8, (rows // 8) * 8))

    out = pl.pallas_call(
        functools.partial(_sums_kernel, rows=rows, tile=tile, minor=minor),
        out_shape=jax.ShapeDtypeStruct((1, 1), f32),
        in_specs=[pl.BlockSpec(memory_space=pltpu.MemorySpace.HBM)] * 3,
        out_specs=pl.BlockSpec(memory_space=pltpu.SMEM),
        scratch_shapes=[pltpu.VMEM((2, tile, minor), f32),
                        pltpu.VMEM((2, tile, minor), f32),
                        pltpu.VMEM((2, tile, minor), f32),
                        pltpu.VMEM((8, minor), f32),
                        pltpu.VMEM((8, minor), f32),
                        pltpu.SemaphoreType.DMA((3, 2))],
        cost_estimate=pl.CostEstimate(
            flops=int(5 * delta_norm.size), transcendentals=0,
            bytes_accessed=int(12 * delta_norm.size)),
    )(delta_norm, logstd, M_obs)
    return out[0, 0]
